# Initial kernel scaffold; baseline (speedup 1.0000x reference)
import jax
import jax.numpy as jnp
from jax.experimental import pallas as pl


def kernel(xyz1, xyz2):
    raise NotImplementedError("write your pallas kernel here")


# speedup vs baseline: 1.9509x; 1.9509x over previous
import jax
import jax.numpy as jnp
from jax.experimental import pallas as pl

_NB = 256


def _chamfer_body(x1_ref, x2t_ref, d1_ref, i1_ref, d2_ref, i2_ref):
    i = pl.program_id(1)
    x1 = x1_ref[0]
    x2t = x2t_ref[0]
    m = x2t.shape[1]

    ab = jax.lax.dot_general(
        x1, x2t, dimension_numbers=(((1,), (0,)), ((), ())),
        preferred_element_type=jnp.float32)
    aa = jnp.sum(x1 * x1, axis=1, keepdims=True)
    bb = jnp.sum(x2t * x2t, axis=0, keepdims=True)
    d = jnp.maximum(aa + bb - 2.0 * ab, 0.0)

    rmin = jnp.min(d, axis=1, keepdims=True)
    lane = jax.lax.broadcasted_iota(jnp.int32, d.shape, 1)
    ridx = jnp.min(jnp.where(d == rmin, lane, jnp.int32(m)),
                   axis=1, keepdims=True)
    d1_ref[0] = rmin
    i1_ref[0] = ridx

    cmin = jnp.min(d, axis=0, keepdims=True)
    row = jax.lax.broadcasted_iota(jnp.int32, d.shape, 0) + i * _NB
    cidx = jnp.min(jnp.where(d == cmin, row, jnp.int32(1 << 30)),
                   axis=0, keepdims=True)

    @pl.when(i == 0)
    def _():
        d2_ref[0] = cmin
        i2_ref[0] = cidx

    @pl.when(i != 0)
    def _():
        prev_d = d2_ref[0]
        prev_i = i2_ref[0]
        take = cmin < prev_d
        d2_ref[0] = jnp.where(take, cmin, prev_d)
        i2_ref[0] = jnp.where(take, cidx, prev_i)


def kernel(xyz1, xyz2):
    b, n, _ = xyz1.shape
    m = xyz2.shape[1]
    x2t = jnp.transpose(xyz2, (0, 2, 1))

    grid = (b, n // _NB)
    d1, i1, d2, i2 = pl.pallas_call(
        _chamfer_body,
        grid=grid,
        in_specs=[
            pl.BlockSpec((1, _NB, 3), lambda bi, ti: (bi, ti, 0)),
            pl.BlockSpec((1, 3, m), lambda bi, ti: (bi, 0, 0)),
        ],
        out_specs=[
            pl.BlockSpec((1, _NB, 1), lambda bi, ti: (bi, ti, 0)),
            pl.BlockSpec((1, _NB, 1), lambda bi, ti: (bi, ti, 0)),
            pl.BlockSpec((1, 1, m), lambda bi, ti: (bi, 0, 0)),
            pl.BlockSpec((1, 1, m), lambda bi, ti: (bi, 0, 0)),
        ],
        out_shape=[
            jax.ShapeDtypeStruct((b, n, 1), jnp.float32),
            jax.ShapeDtypeStruct((b, n, 1), jnp.int32),
            jax.ShapeDtypeStruct((b, 1, m), jnp.float32),
            jax.ShapeDtypeStruct((b, 1, m), jnp.int32),
        ],
    )(xyz1, x2t)

    return (d1[:, :, 0], d2[:, 0, :], i1[:, :, 0], i2[:, 0, :])
